# baseline (speedup 1.0000x reference)
import functools

import jax
import jax.numpy as jnp
from jax import lax
from jax.experimental import pallas as pl
from jax.experimental.pallas import tpu as pltpu
from jax.experimental.pallas import tpu_sc as plsc

f32 = jnp.float32
i32 = jnp.int32

VOCAB = 65
T = 8
KSTR = 128
NKEY = T * KSTR
NCT = 9
LSECS = 1
NTOK = 1024
NC = 2
NS = 16
NW = NC * NS
NUNIT = 18
TPU_ = 512


def _tc_body(tok_t_ref, pos_ref, w_ref, b_ref, ctab_ref):
    W = w_ref[...]
    LtT = lax.dot_general(W, tok_t_ref[...], (((0,), (0,)), ((), ())),
                          preferred_element_type=f32,
                          precision=lax.Precision.HIGHEST)
    LpT = lax.dot_general(W, pos_ref[...], (((0,), (1,)), ((), ())),
                          preferred_element_type=f32,
                          precision=lax.Precision.HIGHEST)
    LtT = LtT + jnp.reshape(b_ref[...], (VOCAB, 1))
    for t in range(T):
        blk = LtT + LpT[:, t:t + 1]
        m = jnp.max(blk, axis=0, keepdims=True)
        s = jnp.sum(jnp.exp(blk - m), axis=0, keepdims=True)
        for ct in range(8):
            ctab_ref[ct, t, :, :VOCAB] = blk[ct * 8:ct * 8 + 8, :]
        ctab_ref[8, t, 0:1, :VOCAB] = blk[VOCAB - 1:VOCAB, :]
        ctab_ref[8, t, 1:2, :VOCAB] = m + jnp.log(s)


_tc_tables = pl.pallas_call(
    _tc_body,
    out_shape=jax.ShapeDtypeStruct((NCT, T, 8, KSTR), f32),
)


@functools.partial(
    pl.kernel,
    mesh=plsc.VectorSubcoreMesh(core_axis_name="c", subcore_axis_name="s"),
    out_type=jax.ShapeDtypeStruct((NCT + 1, T, 8, KSTR), f32),
    scratch_types=[
        pltpu.VMEM((NTOK,), i32),
        pltpu.VMEM((NTOK,), i32),
        pltpu.VMEM((TPU_,), i32),
        pltpu.VMEM((TPU_,), i32),
        pltpu.VMEM((T, 8, KSTR), f32),
        pltpu.VMEM((4, 8, KSTR), f32),
        pltpu.VMEM((16,), f32),
        pltpu.SemaphoreType.DMA,
        pltpu.SemaphoreType.DMA,
        pltpu.SemaphoreType.DMA,
    ],
    compiler_params=pltpu.CompilerParams(
        needs_layout_passes=False, use_tc_tiling_on_sc=False),
)
def _sc_kernel(ctab, idxf, tgtf, out,
               idxm_v, tgtm_v, keys_v, tgt_v, slab_v, outb_v, acc_ref,
               sem, sem2, sem3):
    cid = lax.axis_index("c")
    sid = lax.axis_index("s")
    wid = sid * NC + cid
    ct = lax.div(wid, 2)
    half = lax.rem(wid, 2)
    acc_ref[...] = jnp.zeros((16,), f32)

    @pl.when(wid < NUNIT)
    def _():
        cp_slab = pltpu.async_copy(ctab.at[ct], slab_v, sem)
        cp_idx = pltpu.async_copy(idxf, idxm_v, sem2)
        cp_tgt = pltpu.async_copy(tgtf, tgtm_v, sem3)

        lane = jnp.arange(16, dtype=i32)
        tpos = lax.bitwise_and(lane, T - 1)
        roff = half * (TPU_ // T) + lax.shift_right_logical(lane, 3)
        cp_idx.wait()
        cp_tgt.wait()
        for k in range(TPU_ // 16):
            perm = tpos * KSTR + (roff + 2 * k)
            iv = plsc.load_gather(idxm_v, [perm])
            keys_v[pl.ds(k * 16, 16)] = tpos * KSTR + iv
            tgt_v[pl.ds(k * 16, 16)] = plsc.load_gather(tgtm_v, [perm])

        cp_slab.wait()
        acc = jnp.zeros((16,), f32)
        for cs in range(8):
            c = ct * 8 + cs
            cs_vec = jnp.full((16,), cs, dtype=i32)
            for k in range(TPU_ // 16):
                keys = keys_v[pl.ds(k * 16, 16)]
                kt = lax.shift_right_logical(keys, 7)
                ks = lax.bitwise_and(keys, KSTR - 1)
                vals = plsc.load_gather(slab_v, [kt, cs_vec, ks])
                outb_v[k // 8, cs, pl.ds((k % 8) * 16, 16)] = vals
                tg = tgt_v[pl.ds(k * 16, 16)]
                acc = acc - jnp.where(tg == c, vals, jnp.float32(0.0))

        @pl.when(ct == NCT - 1)
        def _():
            ls = jnp.zeros((16,), f32)
            one = jnp.full((16,), LSECS, dtype=i32)
            for k in range(TPU_ // 16):
                keys = keys_v[pl.ds(k * 16, 16)]
                kt = lax.shift_right_logical(keys, 7)
                ks = lax.bitwise_and(keys, KSTR - 1)
                ls = ls + plsc.load_gather(slab_v, [kt, one, ks])
            acc_ref[...] = ls

        cp_out = pltpu.async_copy(
            outb_v, out.at[ct, pl.ds(half * 4, 4)], sem)
        acc_ref[...] = (acc_ref[...] + acc) * jnp.float32(1.0 / NTOK)
        cp_out.wait()

    pltpu.sync_copy(
        acc_ref,
        out.at[NCT, 0, lax.div(wid, 8), pl.ds(lax.rem(wid, 8) * 16, 16)])


def kernel(idx, targets, tok_table, pos_table, W, b):
    ctab = _tc_tables(tok_table.astype(f32).T, pos_table.astype(f32),
                      W.astype(f32), b.astype(f32))

    idxT = idx.astype(i32).T.reshape(-1)
    tgtT = targets.astype(i32).T.reshape(-1)
    out_raw = _sc_kernel(ctab, idxT, tgtT)

    logits = out_raw[:NCT].transpose(0, 2, 1, 3).reshape(NCT * 8, NKEY)
    logits = logits[:VOCAB].T
    loss = jnp.sum(out_raw[NCT, 0, 0:4])
    return (logits, loss)


# speedup vs baseline: 1.3311x; 1.3311x over previous
import functools

import jax
import jax.numpy as jnp
from jax import lax
from jax.experimental import pallas as pl
from jax.experimental.pallas import tpu as pltpu
from jax.experimental.pallas import tpu_sc as plsc

f32 = jnp.float32
i32 = jnp.int32

VOCAB = 65
T = 8
ROWB = 72
NKEY = ROWB * T
DPAD = 128
LSECOL = 65
NTOK = 1024
NC = 2
NS = 16
NW = NC * NS
BPT = NTOK // NW


def _tc_body(tok_t_ref, pos_ref, w_ref, b_ref, ctab_ref):
    W = w_ref[...]
    Lt = lax.dot_general(tok_t_ref[...], W, (((0,), (0,)), ((), ())),
                         preferred_element_type=f32,
                         precision=lax.Precision.HIGHEST)
    Lt = Lt + b_ref[...]
    Lp = jnp.dot(pos_ref[...], W, preferred_element_type=f32,
                 precision=lax.Precision.HIGHEST)
    for t in range(T):
        blk = Lt + Lp[t:t + 1, :]
        m = jnp.max(blk, axis=1, keepdims=True)
        s = jnp.sum(jnp.exp(blk - m), axis=1, keepdims=True)
        ctab_ref[pl.ds(t * ROWB, VOCAB), :VOCAB] = blk
        ctab_ref[pl.ds(t * ROWB, VOCAB), LSECOL:LSECOL + 1] = m + jnp.log(s)


_tc_tables = pl.pallas_call(
    _tc_body,
    out_shape=jax.ShapeDtypeStruct((NKEY, DPAD), f32),
)


@functools.partial(
    pl.kernel,
    mesh=plsc.VectorSubcoreMesh(core_axis_name="c", subcore_axis_name="s"),
    out_type=(
        jax.ShapeDtypeStruct((NTOK, DPAD), f32),
        jax.ShapeDtypeStruct((NW, 16), f32),
    ),
    scratch_types=[
        pltpu.VMEM((BPT,), i32),
        pltpu.VMEM((BPT,), i32),
        pltpu.VMEM((BPT,), i32),
        pltpu.VMEM((BPT,), i32),
        pltpu.VMEM((BPT, DPAD), f32),
        pltpu.VMEM((16,), f32),
        pltpu.SemaphoreType.DMA,
        pltpu.SemaphoreType.DMA,
        pltpu.SemaphoreType.DMA,
    ],
    compiler_params=pltpu.CompilerParams(
        needs_layout_passes=False, use_tc_tiling_on_sc=False),
)
def _sc_kernel(ctab, idxf, tgtf, out, lpart,
               idx_v, tgt_v, keys_v, orow_v, rows_v, acc_ref,
               sem, sem2, sem3):
    cid = lax.axis_index("c")
    sid = lax.axis_index("s")
    wid = sid * NC + cid
    tpos = lax.div(wid, 4)
    rbase = 32 * lax.rem(wid, 4)

    cp_idx = pltpu.async_copy(idxf.at[pl.ds(wid * BPT, BPT)], idx_v, sem)
    cp_tgt = pltpu.async_copy(tgtf.at[pl.ds(wid * BPT, BPT)], tgt_v, sem2)
    cp_idx.wait()

    lane = jnp.arange(16, dtype=i32)
    for c in range(BPT // 16):
        keys_v[pl.ds(c * 16, 16)] = tpos * ROWB + idx_v[pl.ds(c * 16, 16)]
        orow_v[pl.ds(c * 16, 16)] = (rbase + c * 16 + lane) * T + tpos

    pltpu.async_copy(ctab.at[keys_v], rows_v, sem3).wait()
    cp_out = pltpu.async_copy(rows_v, out.at[orow_v], sem3)

    cp_tgt.wait()
    lsecol = jnp.full((16,), LSECOL, dtype=i32)
    acc = jnp.zeros((16,), f32)
    for c in range(BPT // 16):
        rowi = lane + c * 16
        lg = plsc.load_gather(rows_v, [rowi, lsecol])
        picked = plsc.load_gather(rows_v, [rowi, tgt_v[pl.ds(c * 16, 16)]])
        acc = acc + (lg - picked)
    acc_ref[...] = acc * jnp.float32(1.0 / NTOK)
    pltpu.sync_copy(acc_ref, lpart.at[wid])
    cp_out.wait()


def kernel(idx, targets, tok_table, pos_table, W, b):
    V = tok_table.shape[0]
    ctab = _tc_tables(tok_table.astype(f32).T, pos_table.astype(f32),
                      W.astype(f32), b.astype(f32))

    idxT = idx.astype(i32).T.reshape(-1)
    tgtT = targets.astype(i32).T.reshape(-1)
    out_pad, lpart = _sc_kernel(ctab, idxT, tgtT)

    logits = out_pad[:, :V]
    loss = jnp.sum(lpart)
    return (logits, loss)
